# baseline (speedup 1.0000x reference)
import functools

import jax
import jax.numpy as jnp
from jax import lax
from jax.experimental import pallas as pl
from jax.experimental.pallas import tpu as pltpu
from jax.experimental.pallas import tpu_sc as plsc

_LANES = 16
_WCHUNK = 7
_BG = 8


def _routing_copy(tin, routing_flat, h0):
    H, W, B, C = tin.shape
    R = 8
    rw = C // R
    n_groups = B // _BG
    n_wchunks = W // _WCHUNK
    n_workers = 32 // n_groups
    HS = H - h0

    info = plsc.get_sparse_core_info()
    NC = info.num_cores

    mesh = plsc.VectorSubcoreMesh(core_axis_name="c", subcore_axis_name="s")

    @functools.partial(
        pl.kernel,
        mesh=mesh,
        out_type=jax.ShapeDtypeStruct((HS, W, B, rw), jnp.float32),
        scratch_types=[
            pltpu.VMEM((B * R + _LANES,), jnp.float32),
            pltpu.VMEM((_WCHUNK, _BG, C), jnp.float32),
            pltpu.VMEM((_WCHUNK, _BG, C), jnp.float32),
            pltpu.VMEM((_BG, rw), jnp.float32),
            pltpu.VMEM((_BG, rw), jnp.float32),
            pltpu.SemaphoreType.DMA,
            pltpu.SemaphoreType.DMA,
            pltpu.SemaphoreType.DMA,
            pltpu.SemaphoreType.DMA,
            pltpu.SemaphoreType.DMA,
            pltpu.SemaphoreType.DMA,
        ],
        compiler_params=pltpu.CompilerParams(
            needs_layout_passes=False, skip_device_barrier=True
        ),
    )
    def k(inp_hbm, rout_hbm, out_hbm, rout_v, in_v0, in_v1, out_v0,
          out_v1, isem0a, isem0b, isem1a, isem1b, osem0, osem1):
        in_bufs = (in_v0, in_v1)
        in_sems = ((isem0a, isem0b), (isem1a, isem1b))
        out_bufs = (out_v0, out_v1)
        out_sems = (osem0, osem1)

        wid = lax.axis_index("s") * NC + lax.axis_index("c")
        g = wid % n_groups
        kw = wid // n_groups
        g8 = pl.multiple_of(g * _BG, _BG)
        pltpu.sync_copy(rout_hbm, rout_v)

        lane = lax.broadcasted_iota(jnp.int32, (_LANES,), 0)
        roff = []
        for bi in range(_BG):
            boff = pl.multiple_of((g8 + bi) * R, R)
            v = rout_v[pl.ds(boff, _LANES)]
            v = jnp.where(lane < R, v, -jnp.inf)
            vmax = jnp.max(v)
            r = jnp.min(jnp.where(v == vmax, lane, jnp.int32(_LANES)))
            roff.append(pl.multiple_of(r * rw, _LANES))

        n_chunks = HS * n_wchunks
        per_w = n_chunks // n_workers
        ci_lo = kw * per_w
        ci_hi = ci_lo + per_w

        half = C // 2

        def in_copies(ci, buf, sems):
            h = h0 + ci // n_wchunks
            w0 = (ci % n_wchunks) * _WCHUNK
            return [
                pltpu.make_async_copy(
                    inp_hbm.at[h, pl.ds(w0, _WCHUNK), pl.ds(g8, _BG),
                               pl.ds(q * half, half)],
                    buf.at[:, :, pl.ds(q * half, half)],
                    sems[q],
                )
                for q in range(2)
            ]

        def start_in(ci, buf, sems):
            for cpy in in_copies(ci, buf, sems):
                cpy.start()

        def wait_in(ci, buf, sems):
            for cpy in in_copies(ci, buf, sems):
                cpy.wait()

        start_in(ci_lo, in_bufs[0], in_sems[0])

        def pair(cp, _):
            for p in range(2):
                ci = cp * 2 + p
                @pl.when(ci + 1 < ci_hi)
                def _():
                    start_in(ci + 1, in_bufs[1 - p], in_sems[1 - p])

                wait_in(ci, in_bufs[p], in_sems[p])
                h = ci // n_wchunks
                w0 = (ci % n_wchunks) * _WCHUNK
                in_v = in_bufs[p]

                def do_w(w, parity, first_use):
                    ob = out_bufs[parity]
                    osem = out_sems[parity]
                    if first_use:
                        @pl.when(ci > ci_lo)
                        def _():
                            pltpu.make_async_copy(
                                ob, out_hbm.at[0, 0, pl.ds(0, _BG), :],
                                osem,
                            ).wait()
                    else:
                        pltpu.make_async_copy(
                            ob, out_hbm.at[0, 0, pl.ds(0, _BG), :], osem
                        ).wait()
                    for bi in range(_BG):
                        for j in range(rw // _LANES):
                            ob[bi, pl.ds(j * _LANES, _LANES)] = in_v[
                                w, bi,
                                pl.ds(roff[bi] + j * _LANES, _LANES)
                            ]
                    pltpu.make_async_copy(
                        ob, out_hbm.at[h, w0 + w, pl.ds(g8, _BG), :], osem
                    ).start()

                def wpair(q, _):
                    do_w(q * 2, 0, False)
                    do_w(q * 2 + 1, 1, False)
                    return 0

                do_w(0, 0, True)
                do_w(1, 1, True)
                lax.fori_loop(1, _WCHUNK // 2, wpair, 0)
                do_w(_WCHUNK - 1, 0, False)
            return 0

        lax.fori_loop(ci_lo // 2, ci_hi // 2, pair, 0)
        for q in range(2):
            pltpu.make_async_copy(
                out_bufs[q], out_hbm.at[0, 0, pl.ds(0, _BG), :], out_sems[q]
            ).wait()

    return k(tin, routing_flat)


def _routing_copy_tc(tin, routing, ht):
    H, W, B, C = tin.shape
    R = routing.shape[-1]
    rw = C // R

    def body(rout_ref, in_ref, out_ref):
        rr = rout_ref[...]
        m = jnp.max(rr, axis=1, keepdims=True)
        i8 = lax.broadcasted_iota(jnp.int32, (B, R), 1)
        route = jnp.min(jnp.where(rr >= m, i8, R), axis=1)
        x = in_ref[...]
        acc = x[..., 0:rw]
        for r in range(1, R):
            seg = x[..., r * rw:(r + 1) * rw]
            acc = jnp.where((route == r)[None, None, :, None], seg, acc)
        out_ref[...] = acc

    return pl.pallas_call(
        body,
        grid=(ht,),
        in_specs=[
            pl.BlockSpec((B, R), lambda h: (0, 0)),
            pl.BlockSpec((1, W, B, C), lambda h: (h, 0, 0, 0)),
        ],
        out_specs=pl.BlockSpec((1, W, B, rw), lambda h: (h, 0, 0, 0)),
        out_shape=jax.ShapeDtypeStruct((ht, W, B, rw), jnp.float32),
    )(routing, tin)


_HT = 8


def kernel(inputs, routing_inputs):
    B, R = routing_inputs.shape
    H = inputs.shape[1]
    routing_flat = jnp.pad(
        jnp.reshape(routing_inputs, (B * R,)), (0, _LANES),
        constant_values=-jnp.inf,
    )
    tin = jnp.transpose(inputs, (1, 2, 0, 3))
    sc_out = _routing_copy(tin, routing_flat, _HT)
    tc_out = _routing_copy_tc(tin, routing_inputs, _HT)
    tout = jnp.zeros((H,) + sc_out.shape[1:], jnp.float32)
    tout = lax.dynamic_update_slice(tout, tc_out, (0, 0, 0, 0))
    tout = lax.dynamic_update_slice(tout, sc_out, (_HT, 0, 0, 0))
    return jnp.transpose(tout, (2, 0, 1, 3))


# speedup vs baseline: 1.1328x; 1.1328x over previous
import functools

import jax
import jax.numpy as jnp
from jax import lax
from jax.experimental import pallas as pl
from jax.experimental.pallas import tpu as pltpu
from jax.experimental.pallas import tpu_sc as plsc

_LANES = 16
_WCHUNK = 7
_BG = 8


def _routing_copy(tin, routing_flat):
    H, W, B, C = tin.shape
    R = 8
    rw = C // R
    n_groups = B // _BG
    n_wchunks = W // _WCHUNK
    n_workers = 32 // n_groups

    info = plsc.get_sparse_core_info()
    NC = info.num_cores

    mesh = plsc.VectorSubcoreMesh(core_axis_name="c", subcore_axis_name="s")

    @functools.partial(
        pl.kernel,
        mesh=mesh,
        out_type=jax.ShapeDtypeStruct((H, W, B, rw), jnp.float32),
        scratch_types=[
            pltpu.VMEM((B * R + _LANES,), jnp.float32),
            pltpu.VMEM((_WCHUNK, _BG, C), jnp.float32),
            pltpu.VMEM((_WCHUNK, _BG, C), jnp.float32),
            pltpu.VMEM((_BG, rw), jnp.float32),
            pltpu.VMEM((_BG, rw), jnp.float32),
            pltpu.SemaphoreType.DMA,
            pltpu.SemaphoreType.DMA,
            pltpu.SemaphoreType.DMA,
            pltpu.SemaphoreType.DMA,
            pltpu.SemaphoreType.DMA,
            pltpu.SemaphoreType.DMA,
        ],
        compiler_params=pltpu.CompilerParams(
            needs_layout_passes=False, skip_device_barrier=True
        ),
    )
    def k(inp_hbm, rout_hbm, out_hbm, rout_v, in_v0, in_v1, out_v0,
          out_v1, isem0a, isem0b, isem1a, isem1b, osem0, osem1):
        in_bufs = (in_v0, in_v1)
        in_sems = ((isem0a, isem0b), (isem1a, isem1b))
        out_bufs = (out_v0, out_v1)
        out_sems = (osem0, osem1)

        wid = lax.axis_index("s") * NC + lax.axis_index("c")
        g = wid % n_groups
        kw = wid // n_groups
        g8 = pl.multiple_of(g * _BG, _BG)
        pltpu.sync_copy(rout_hbm, rout_v)

        lane = lax.broadcasted_iota(jnp.int32, (_LANES,), 0)
        roff = []
        for bi in range(_BG):
            boff = pl.multiple_of((g8 + bi) * R, R)
            v = rout_v[pl.ds(boff, _LANES)]
            v = jnp.where(lane < R, v, -jnp.inf)
            vmax = jnp.max(v)
            r = jnp.min(jnp.where(v == vmax, lane, jnp.int32(_LANES)))
            roff.append(pl.multiple_of(r * rw, _LANES))

        n_chunks = H * n_wchunks
        per_w = n_chunks // n_workers
        ci_lo = kw * per_w
        ci_hi = ci_lo + per_w

        half = C // 2

        def in_copies(ci, buf, sems):
            h = ci // n_wchunks
            w0 = (ci % n_wchunks) * _WCHUNK
            return [
                pltpu.make_async_copy(
                    inp_hbm.at[h, pl.ds(w0, _WCHUNK), pl.ds(g8, _BG),
                               pl.ds(q * half, half)],
                    buf.at[:, :, pl.ds(q * half, half)],
                    sems[q],
                )
                for q in range(2)
            ]

        def start_in(ci, buf, sems):
            for cpy in in_copies(ci, buf, sems):
                cpy.start()

        def wait_in(ci, buf, sems):
            for cpy in in_copies(ci, buf, sems):
                cpy.wait()

        start_in(ci_lo, in_bufs[0], in_sems[0])

        def pair(cp, _):
            for p in range(2):
                ci = cp * 2 + p
                @pl.when(ci + 1 < ci_hi)
                def _():
                    start_in(ci + 1, in_bufs[1 - p], in_sems[1 - p])

                wait_in(ci, in_bufs[p], in_sems[p])
                h = ci // n_wchunks
                w0 = (ci % n_wchunks) * _WCHUNK
                in_v = in_bufs[p]

                def do_w(w, parity, first_use):
                    ob = out_bufs[parity]
                    osem = out_sems[parity]
                    if first_use:
                        @pl.when(ci > ci_lo)
                        def _():
                            pltpu.make_async_copy(
                                ob, out_hbm.at[0, 0, pl.ds(0, _BG), :],
                                osem,
                            ).wait()
                    else:
                        pltpu.make_async_copy(
                            ob, out_hbm.at[0, 0, pl.ds(0, _BG), :], osem
                        ).wait()
                    for bi in range(_BG):
                        for j in range(rw // _LANES):
                            ob[bi, pl.ds(j * _LANES, _LANES)] = in_v[
                                w, bi,
                                pl.ds(roff[bi] + j * _LANES, _LANES)
                            ]
                    pltpu.make_async_copy(
                        ob, out_hbm.at[h, w0 + w, pl.ds(g8, _BG), :], osem
                    ).start()

                def wpair(q, _):
                    do_w(q * 2, 0, False)
                    do_w(q * 2 + 1, 1, False)
                    return 0

                do_w(0, 0, True)
                do_w(1, 1, True)
                lax.fori_loop(1, _WCHUNK // 2, wpair, 0)
                do_w(_WCHUNK - 1, 0, False)
            return 0

        lax.fori_loop(ci_lo // 2, ci_hi // 2, pair, 0)
        for q in range(2):
            pltpu.make_async_copy(
                out_bufs[q], out_hbm.at[0, 0, pl.ds(0, _BG), :], out_sems[q]
            ).wait()

    return k(tin, routing_flat)


def kernel(inputs, routing_inputs):
    B, R = routing_inputs.shape
    routing_flat = jnp.pad(
        jnp.reshape(routing_inputs, (B * R,)), (0, _LANES),
        constant_values=-jnp.inf,
    )
    tin = jnp.transpose(inputs, (1, 2, 0, 3))
    tout = _routing_copy(tin, routing_flat)
    return jnp.transpose(tout, (2, 0, 1, 3))
